# baseline (speedup 1.0000x reference)
import jax
import jax.numpy as jnp
from jax.experimental import pallas as pl
from jax.experimental.pallas import tpu as pltpu

_NCLS = 1000
_MAXP = 100
_FDIM = 128
_CB = 8
_K = _NCLS // _CB


def _body(cid_ref, feat_hbm, protos_hbm, counts_hbm, protos_out, counts_out,
          stage, featv, countsv, sems, sem_st, sem_f, sem_cin, sem_cout,
          sem_pout):
    cid = cid_ref[0]
    c_star = cid // _CB
    local = cid - c_star * _CB

    def chunk_copy(k):
        return pltpu.make_async_copy(
            protos_hbm.at[pl.ds(k * _CB, _CB)],
            protos_out.at[pl.ds(k * _CB, _CB)],
            sems.at[k])

    stage_in = pltpu.make_async_copy(
        protos_hbm.at[pl.ds(c_star * _CB, _CB)], stage, sem_st)
    stage_out = pltpu.make_async_copy(
        stage, protos_out.at[pl.ds(c_star * _CB, _CB)], sem_pout)
    feat_in = pltpu.make_async_copy(feat_hbm.at[pl.ds(0, 104)], featv, sem_f)
    counts_in = pltpu.make_async_copy(counts_hbm, countsv, sem_cin)
    counts_wr = pltpu.make_async_copy(countsv, counts_out, sem_cout)

    stage_in.start()
    feat_in.start()
    counts_in.start()
    for k in range(_K):
        @pl.when(k != c_star)
        def _():
            chunk_copy(k).start()

    feat_in.wait()
    f = featv[...]
    norm = jnp.sqrt(jnp.sum(f * f, axis=1, keepdims=True))
    fn = f / jnp.maximum(norm, 1e-12)

    stage_in.wait()
    stage[pl.ds(local, 1)] = fn[:_MAXP][None]
    stage_out.start()

    counts_in.wait()
    countsv[pl.ds(cid, 1)] = jnp.ones((1, _MAXP), jnp.int32)
    counts_wr.start()

    for k in range(_K):
        @pl.when(k != c_star)
        def _():
            chunk_copy(k).wait()
    stage_out.wait()
    counts_wr.wait()


def kernel(features, prototypes, counts, class_id):
    cid = jnp.atleast_1d(jnp.asarray(class_id, jnp.int32))
    grid_spec = pltpu.PrefetchScalarGridSpec(
        num_scalar_prefetch=1,
        grid=(1,),
        in_specs=[pl.BlockSpec(memory_space=pltpu.MemorySpace.HBM)] * 3,
        out_specs=[pl.BlockSpec(memory_space=pltpu.MemorySpace.HBM)] * 2,
        scratch_shapes=[
            pltpu.VMEM((_CB, _MAXP, _FDIM), jnp.float32),
            pltpu.VMEM((104, _FDIM), jnp.float32),
            pltpu.VMEM((_NCLS, _MAXP), jnp.int32),
            pltpu.SemaphoreType.DMA((_K,)),
            pltpu.SemaphoreType.DMA,
            pltpu.SemaphoreType.DMA,
            pltpu.SemaphoreType.DMA,
            pltpu.SemaphoreType.DMA,
            pltpu.SemaphoreType.DMA,
        ],
    )
    return pl.pallas_call(
        _body,
        grid_spec=grid_spec,
        out_shape=(
            jax.ShapeDtypeStruct((_NCLS, _MAXP, _FDIM), jnp.float32),
            jax.ShapeDtypeStruct((_NCLS, _MAXP), jnp.int32),
        ),
        compiler_params=pltpu.CompilerParams(
            dimension_semantics=("arbitrary",),
        ),
    )(cid, features, prototypes, counts)
